# Initial kernel scaffold; baseline (speedup 1.0000x reference)
import jax
import jax.numpy as jnp
from jax.experimental import pallas as pl


def kernel(Xi, Xv, fm_first, fm_second, bias, W_att, b_att, H, P):
    raise NotImplementedError("write your pallas kernel here")


# speedup vs baseline: 1.5525x; 1.5525x over previous
import functools

import jax
import jax.numpy as jnp
from jax import lax
from jax.experimental import pallas as pl
from jax.experimental.pallas import tpu as pltpu
from jax.experimental.pallas import tpu_sc as plsc

B = 4096
F = 26
V = 100000
E = 16
A = 16

NW = 32
CHUNK = 128
PER_W = (B * F) // NW
CHUNKS_W = PER_W // CHUNK
BS = 8
RB = BS * F
GRID = B // BS


def _sc_gather(fm2_flat, fm1_wide, gidx3d, g1div3d):
    mesh = plsc.VectorSubcoreMesh(core_axis_name="c", subcore_axis_name="s")

    @functools.partial(
        pl.kernel,
        mesh=mesh,
        out_type=[
            jax.ShapeDtypeStruct((B * F, E), jnp.float32),
            jax.ShapeDtypeStruct((B * F, 16), jnp.float32),
        ],
        scratch_types=[
            pltpu.VMEM((CHUNKS_W, CHUNK), jnp.int32),
            pltpu.VMEM((CHUNKS_W, CHUNK), jnp.int32),
            pltpu.VMEM((PER_W, E), jnp.float32),
            pltpu.VMEM((PER_W, 16), jnp.float32),
            pltpu.SemaphoreType.DMA,
            pltpu.SemaphoreType.DMA,
        ],
        compiler_params=pltpu.CompilerParams(use_tc_tiling_on_sc=False),
    )
    def k(fm2_hbm, fm1_hbm, idx_hbm, div_hbm, e2_hbm, e1_hbm,
          idx_v, div_v, rows_v, rows1_v, sem2, sem1):
        wid = lax.axis_index("s") * 2 + lax.axis_index("c")
        base = wid * PER_W
        pltpu.sync_copy(idx_hbm.at[wid], idx_v)
        pltpu.sync_copy(div_hbm.at[wid], div_v)
        d2, d1 = [], []
        for j in range(CHUNKS_W):
            d2.append(pltpu.async_copy(
                fm2_hbm.at[idx_v.at[j]],
                rows_v.at[pl.ds(j * CHUNK, CHUNK)], sem2))
            d1.append(pltpu.async_copy(
                fm1_hbm.at[div_v.at[j]],
                rows1_v.at[pl.ds(j * CHUNK, CHUNK)], sem1))
        for d in d2:
            d.wait()
        for d in d1:
            d.wait()
        pltpu.sync_copy(rows_v, e2_hbm.at[pl.ds(base, PER_W)])
        pltpu.sync_copy(rows1_v, e1_hbm.at[pl.ds(base, PER_W)])

    return k(fm2_flat, fm1_wide, gidx3d, g1div3d)


def _tc_body(e2_ref, e1_ref, rem_ref, xv_ref, w_att_ref, h_ref, b_att_ref,
             p_ref, bias_ref, o_ref):
    xv = xv_ref[...]
    x = e2_ref[...] * xv
    hv = h_ref[...]
    w = jnp.dot(hv, w_att_ref[...],
                preferred_element_type=jnp.float32)
    c = jnp.sum(b_att_ref[...] * hv)
    aw = jnp.concatenate([x * w, x * p_ref[...]], axis=0)
    g = lax.dot_general(aw, x, (((1,), (1,)), ((), ())),
                        preferred_element_type=jnp.float32)
    gw = g[:RB, :]
    gp = g[RB:, :]
    r = lax.broadcasted_iota(jnp.int32, (RB, RB), 0)
    col = lax.broadcasted_iota(jnp.int32, (RB, RB), 1)
    mask = ((r // F) == (col // F)) & ((r % F) < (col % F))
    eu = jnp.where(mask, jnp.exp(gw + c), 0.0)
    r1 = jnp.sum(eu, axis=1, keepdims=True)
    r2 = jnp.sum(eu * gp, axis=1, keepdims=True)
    rs = lax.broadcasted_iota(jnp.int32, (BS, RB), 0)
    cs = lax.broadcasted_iota(jnp.int32, (BS, RB), 1)
    sm = (rs == (cs // F)).astype(jnp.float32)
    s1 = jnp.dot(sm, r1, preferred_element_type=jnp.float32)
    s2 = jnp.dot(sm, r2, preferred_element_type=jnp.float32)
    lane = lax.broadcasted_iota(jnp.int32, (RB, 16), 1)
    e1col = jnp.sum(jnp.where(lane == rem_ref[...], e1_ref[...], 0.0),
                    axis=1, keepdims=True)
    e1s = jnp.dot(sm, e1col * xv,
                  preferred_element_type=jnp.float32)
    o_ref[...] = bias_ref[...] + e1s + s2 / s1


def kernel(Xi, Xv, fm_first, fm_second, bias, W_att, b_att, H, P):
    idx = Xi[:, :, 0].astype(jnp.int32)
    gidx = idx + (jnp.arange(F, dtype=jnp.int32) * V)[None, :]
    gidx3d = gidx.reshape(NW, CHUNKS_W, CHUNK)
    fm2_flat = fm_second.reshape(F * V, E)
    fm1_wide = fm_first.reshape((F * V) // 16, 16)
    g1div3d = (gidx // 16).reshape(NW, CHUNKS_W, CHUNK)
    g1rem = (gidx % 16).reshape(B * F, 1)

    e2g, e1g = _sc_gather(fm2_flat, fm1_wide, gidx3d, g1div3d)

    out2d = pl.pallas_call(
        _tc_body,
        grid=(GRID,),
        in_specs=[
            pl.BlockSpec((RB, E), lambda i: (i, 0)),
            pl.BlockSpec((RB, 16), lambda i: (i, 0)),
            pl.BlockSpec((RB, 1), lambda i: (i, 0)),
            pl.BlockSpec((RB, 1), lambda i: (i, 0)),
            pl.BlockSpec((A, E), lambda i: (0, 0)),
            pl.BlockSpec((1, A), lambda i: (0, 0)),
            pl.BlockSpec((1, A), lambda i: (0, 0)),
            pl.BlockSpec((1, E), lambda i: (0, 0)),
            pl.BlockSpec((1, 1), lambda i: (0, 0)),
        ],
        out_specs=pl.BlockSpec((BS, 1), lambda i: (i, 0)),
        out_shape=jax.ShapeDtypeStruct((B, 1), jnp.float32),
    )(e2g, e1g, g1rem, Xv.reshape(B * F, 1), W_att, H.reshape(1, A),
      b_att.reshape(1, A), P.reshape(1, E), bias.reshape(1, 1))
    return out2d.reshape(B)
